# baseline (speedup 1.0000x reference)
import jax
import jax.numpy as jnp
from jax.experimental import pallas as pl

NUM_QUANTIZERS = 8
CODEBOOK_SIZE = 8192
DIM = 32
B = 32
N = 1024
TOKENS = B * N
BLOCK_T = 256
GRID = TOKENS // BLOCK_T


def _rvq_kernel(x_ref, cbt_ref, cbb_ref, cbsq_ref, idx_ref, r_ref):
    T = x_ref.shape[0]
    K = cbt_ref.shape[1]
    r = x_ref[...]
    iota = jax.lax.broadcasted_iota(jnp.int32, (T, K), 1)
    for q in range(NUM_QUANTIZERS):
        prod = jax.lax.dot_general(
            -2.0 * r, cbt_ref[...], (((1,), (0,)), ((), ())),
            precision=jax.lax.Precision.DEFAULT,
            preferred_element_type=jnp.float32,
        )
        r_sq = jnp.sum(r * r, axis=1, keepdims=True)
        s = (r_sq + prod) + cbsq_ref[...]
        m = jnp.min(s, axis=1, keepdims=True)
        idx = jnp.min(jnp.where(s == m, iota, K), axis=1, keepdims=True)
        onehot = (iota == idx).astype(jnp.float32)
        quant = jax.lax.dot_general(
            onehot, cbb_ref[...], (((1,), (0,)), ((), ())),
            precision=jax.lax.Precision.HIGHEST,
            preferred_element_type=jnp.float32,
        )
        r = r - quant
        idx_ref[:, q] = idx[:, 0]
        r_ref[:, q, :] = r


@jax.jit
def kernel(x, codebook):
    xf = x.reshape(TOKENS, DIM)
    cbt = codebook.T
    cb_sq = jnp.sum(codebook * codebook, axis=-1)[None, :]
    idx, rsteps = pl.pallas_call(
        _rvq_kernel,
        grid=(GRID,),
        in_specs=[
            pl.BlockSpec((BLOCK_T, DIM), lambda i: (i, 0)),
            pl.BlockSpec((DIM, CODEBOOK_SIZE), lambda i: (0, 0)),
            pl.BlockSpec((CODEBOOK_SIZE, DIM), lambda i: (0, 0)),
            pl.BlockSpec((1, CODEBOOK_SIZE), lambda i: (0, 0)),
        ],
        out_specs=[
            pl.BlockSpec((BLOCK_T, NUM_QUANTIZERS), lambda i: (i, 0)),
            pl.BlockSpec((BLOCK_T, NUM_QUANTIZERS, DIM), lambda i: (i, 0, 0)),
        ],
        out_shape=[
            jax.ShapeDtypeStruct((TOKENS, NUM_QUANTIZERS), jnp.int32),
            jax.ShapeDtypeStruct((TOKENS, NUM_QUANTIZERS, DIM), jnp.float32),
        ],
    )(xf, cbt, codebook, cb_sq)
    qout = xf - rsteps[:, NUM_QUANTIZERS - 1, :]
    commit = jnp.mean(rsteps * rsteps, axis=(0, 2))
    return (
        qout.reshape(B, N, DIM),
        idx.reshape(B, N, NUM_QUANTIZERS),
        commit,
    )
